# Initial kernel scaffold; baseline (speedup 1.0000x reference)
import jax
import jax.numpy as jnp
from jax.experimental import pallas as pl


def kernel(species, conv_tensor):
    raise NotImplementedError("write your pallas kernel here")


# speedup vs baseline: 1.7018x; 1.7018x over previous
import functools

import jax
import jax.numpy as jnp
from jax import lax
from jax.experimental import pallas as pl
from jax.experimental.pallas import tpu as pltpu
from jax.experimental.pallas import tpu_sc as plsc

DIM = 128
NC = 2
NS = 16
NW = NC * NS
CHUNK = 128


def _sc_embed(table_hbm, idx_hbm, out_hbm, idx_v, rows_v, sem):
    wid = lax.axis_index("s") * NC + lax.axis_index("c")
    b_per_w = idx_hbm.shape[0] // NW
    n_chunks = b_per_w // CHUNK
    base = wid * b_per_w

    def body(i, carry):
        off = base + i * CHUNK
        pltpu.sync_copy(idx_hbm.at[pl.ds(off, CHUNK)], idx_v)
        pltpu.async_copy(table_hbm.at[idx_v], rows_v, sem).wait()
        pltpu.sync_copy(rows_v, out_hbm.at[pl.ds(off, CHUNK)])
        return carry

    lax.fori_loop(0, n_chunks, body, 0)


def kernel(species, conv_tensor):
    n, m = species.shape
    b = n * m
    idx = species.reshape(b).astype(jnp.int32)

    mesh = plsc.VectorSubcoreMesh(
        core_axis_name="c", subcore_axis_name="s", num_cores=NC, num_subcores=NS
    )
    run = functools.partial(
        pl.kernel,
        mesh=mesh,
        out_type=jax.ShapeDtypeStruct((b, DIM), jnp.float32),
        scratch_types=[
            pltpu.VMEM((CHUNK,), jnp.int32),
            pltpu.VMEM((CHUNK, DIM), jnp.float32),
            pltpu.SemaphoreType.DMA,
        ],
    )(_sc_embed)
    out = run(conv_tensor, idx)
    return out.reshape(n, m, DIM)


# speedup vs baseline: 1.7267x; 1.0147x over previous
import functools

import jax
import jax.numpy as jnp
from jax import lax
from jax.experimental import pallas as pl
from jax.experimental.pallas import tpu as pltpu
from jax.experimental.pallas import tpu_sc as plsc

DIM = 128
NC = 2
NS = 16
NW = NC * NS
CHUNK = 128
GROUP = 2
GROWS = GROUP * CHUNK


def _sc_embed(table_hbm, idx_hbm, out_hbm, idx_all, rows_v, sem_g, sem_o):
    wid = lax.axis_index("s") * NC + lax.axis_index("c")
    n_chunks_w = idx_hbm.shape[0] // NW
    n_groups = n_chunks_w // GROUP
    chunk_base = wid * n_chunks_w

    pltpu.sync_copy(idx_hbm.at[pl.ds(chunk_base, n_chunks_w)], idx_all)

    def super_body(sg, carry):
        for p in range(2):
            g = sg * 2 + p
            @pl.when(sg > 0)
            def _():
                pltpu.make_async_copy(
                    out_hbm.at[pl.ds(0, GROWS)], rows_v.at[p], sem_o
                ).wait()

            gathers = []
            for b in range(GROUP):
                gathers.append(pltpu.async_copy(
                    table_hbm.at[idx_all.at[g * GROUP + b]],
                    rows_v.at[p, pl.ds(b * CHUNK, CHUNK)],
                    sem_g,
                ))
            for d in gathers:
                d.wait()
            row0 = (chunk_base + g * GROUP) * CHUNK
            pltpu.async_copy(
                rows_v.at[p], out_hbm.at[pl.ds(row0, GROWS)], sem_o
            )
        return carry

    lax.fori_loop(0, n_groups // 2, super_body, 0)

    for p in range(2):
        pltpu.make_async_copy(
            out_hbm.at[pl.ds(0, GROWS)], rows_v.at[p], sem_o
        ).wait()


def kernel(species, conv_tensor):
    n, m = species.shape
    b = n * m
    idx = species.reshape(b // CHUNK, CHUNK).astype(jnp.int32)

    mesh = plsc.VectorSubcoreMesh(
        core_axis_name="c", subcore_axis_name="s", num_cores=NC, num_subcores=NS
    )
    run = functools.partial(
        pl.kernel,
        mesh=mesh,
        out_type=jax.ShapeDtypeStruct((b, DIM), jnp.float32),
        scratch_types=[
            pltpu.VMEM((b // CHUNK // NW, CHUNK), jnp.int32),
            pltpu.VMEM((2, GROWS, DIM), jnp.float32),
            pltpu.SemaphoreType.DMA,
            pltpu.SemaphoreType.DMA,
        ],
    )(_sc_embed)
    out = run(conv_tensor, idx)
    return out.reshape(n, m, DIM)
